# baseline (speedup 1.0000x reference)
import functools

import numpy as np

import jax
import jax.numpy as jnp
from jax import lax
from jax.experimental import pallas as pl
from jax.experimental.pallas import tpu as pltpu
from jax.experimental.pallas import tpu_sc as plsc

N = 10000
E = 320000
NEG_SLOPE = 0.2

NC = 2
NS = 16
NW = NC * NS
NLOC = 320
N_PAD = NW * NLOC
B_E = 640
NBLK = E // B_E
ROWC = 32

PERM = np.zeros(128, np.int32)
for _f in range(4):
  for _k in range(16):
    PERM[32 * _f + 2 * _k] = 32 * _f + _k
    PERM[32 * _f + 2 * _k + 1] = 32 * _f + 16 + _k


def _tc_proj(x, W, a2, apply_relu):
  n, _ = x.shape
  d_out = W.shape[1]

  def body(x_ref, w_ref, a_ref, h_ref, al_ref):
    xv = x_ref[...]
    if apply_relu:
      xv = jnp.maximum(xv, 0.0)
    h = jnp.dot(xv, w_ref[...], preferred_element_type=jnp.float32)
    h_ref[...] = h
    al_ref[...] = jnp.dot(h, a_ref[...], preferred_element_type=jnp.float32)

  return pl.pallas_call(
      body,
      out_shape=[
          jax.ShapeDtypeStruct((n, d_out), jnp.float32),
          jax.ShapeDtypeStruct((n, 2), jnp.float32),
      ],
  )(x, W, a2)


def _sc_layer(src, dst, asrc, adst, h_pk, d):
  nfb = d // 32
  mesh = plsc.VectorSubcoreMesh(core_axis_name="c", subcore_axis_name="s")

  @functools.partial(
      pl.kernel,
      out_type=jax.ShapeDtypeStruct((N_PAD, d), jnp.float32),
      mesh=mesh,
      compiler_params=pltpu.CompilerParams(needs_layout_passes=False),
      scratch_types=[
          pltpu.VMEM_SHARED((N // 2, 128), jnp.int32),
          pltpu.VMEM_SHARED((N // 2,), jnp.int32),
          pltpu.VMEM((NLOC + 16,), jnp.float32),
          pltpu.VMEM((B_E,), jnp.int32),
          pltpu.VMEM((B_E,), jnp.int32),
          pltpu.VMEM((B_E,), jnp.int32),
          pltpu.VMEM((B_E,), jnp.int32),
          pltpu.VMEM((B_E + 16,), jnp.int32),
          pltpu.VMEM((ROWC,), jnp.int32),
          pltpu.VMEM((ROWC,), jnp.int32),
          pltpu.VMEM((ROWC,), jnp.float32),
          pltpu.VMEM((ROWC, 128), jnp.int32),
          pltpu.VMEM((NLOC, d), jnp.float32),
          pltpu.VMEM((NLOC, 16), jnp.float32),
          pltpu.SemaphoreType.DMA,
          pltpu.SemaphoreType.DMA,
          pltpu.SemaphoreType.DMA,
          pltpu.SemaphoreType.DMA,
      ],
  )
  def k(src_hbm, dst_hbm, asrc_hbm, adst_hbm, hpk_hbm, out_hbm,
        h_sh, asrc_sh, adst_loc, src_a, dst_a, src_b, dst_b, sel_pk,
        idx_rows, apair_chunk, ew_chunk, rows, acc, zacc,
        sem, sem2, sem_s, sem_d):
    s_id = lax.axis_index("s")
    w = s_id * NC + lax.axis_index("c")
    lo = w * NLOC

    @pl.when(s_id < 5)
    def _():
      pltpu.sync_copy(hpk_hbm.at[pl.ds(s_id * 1000, 1000)],
                      h_sh.at[pl.ds(s_id * 1000, 1000)])

    @pl.when(s_id == 5)
    def _():
      pltpu.sync_copy(asrc_hbm, asrc_sh)

    pltpu.sync_copy(adst_hbm.at[pl.ds(lo, NLOC + 16)], adst_loc)
    plsc.subcore_barrier()

    zeros16f = jnp.zeros((16,), jnp.float32)
    zeros16i = jnp.zeros((16,), jnp.int32)

    def zrow(j, carry):
      for f in range(d // 16):
        acc[j, pl.ds(f * 16, 16)] = zeros16f
      zacc[j, :] = zeros16f
      return carry
    lax.fori_loop(0, NLOC, zrow, 0)

    def zsel(i, carry):
      sel_pk[pl.ds(i * 16, 16)] = zeros16i
      return carry
    lax.fori_loop(0, (B_E + 16) // 16, zsel, 0)

    def do_block(base, src_v, dst_v):
      def sel_body(i, cur):
        sv = src_v[pl.ds(i * 16, 16)]
        dv = dst_v[pl.ds(i * 16, 16)]
        m = (dv >= lo) & (dv < lo + NLOC)
        c = jnp.cumsum(m.astype(jnp.int32))
        pos = jnp.where(m, cur + c - 1, B_E + 8)
        plsc.store_scatter(sel_pk, [pos], ((dv - lo) << 15) | sv)
        return cur + c[15]
      cnt = lax.fori_loop(0, B_E // 16, sel_body, jnp.int32(0))

      sel_pk[pl.ds(cnt, 16)] = zeros16i

      def g_body(gi, carry):
        for q in range(ROWC // 16):
          sv = sel_pk[pl.ds(gi * ROWC + q * 16, 16)] & 32767
          idx_rows[pl.ds(q * 16, 16)] = sv >> 1
        pltpu.async_copy(h_sh.at[idx_rows], rows, sem)
        pltpu.async_copy(asrc_sh.at[idx_rows], apair_chunk, sem2)
        pltpu.make_async_copy(h_sh.at[idx_rows], rows, sem).wait()
        pltpu.make_async_copy(asrc_sh.at[idx_rows], apair_chunk, sem2).wait()

        himask = jnp.full((16,), jnp.int32(-65536))
        for q in range(ROWC // 16):
          pk = sel_pk[pl.ds(gi * ROWC + q * 16, 16)]
          raw = apair_chunk[pl.ds(q * 16, 16)]
          asr = jnp.where((pk & 1) == 1,
                          plsc.bitcast(raw & himask, jnp.float32),
                          plsc.bitcast(raw << 16, jnp.float32))
          a = asr + plsc.load_gather(adst_loc, [pk >> 15])
          e = jnp.maximum(a, NEG_SLOPE * a)
          ew_chunk[pl.ds(q * 16, 16)] = jnp.exp(e)

        kmax = jnp.minimum(ROWC, cnt - gi * ROWC)

        def k_body(ki, c2):
          e_idx = gi * ROWC + ki
          s = sel_pk[pl.ds(e_idx, 16)][0]
          j = s >> 15
          colbase = (s & 1) * 64
          sw = ew_chunk[pl.ds(ki, 16)][0]
          for f in range(nfb):
            raw = rows[ki, pl.ds(colbase + f * 16, 16)]
            evn = plsc.bitcast(raw << 16, jnp.float32)
            odd = plsc.bitcast(raw & himask, jnp.float32)
            acc[j, pl.ds(f * 32, 16)] = (
                acc[j, pl.ds(f * 32, 16)] + evn * sw)
            acc[j, pl.ds(f * 32 + 16, 16)] = (
                acc[j, pl.ds(f * 32 + 16, 16)] + odd * sw)
          zacc[j, :] = zacc[j, :] + sw
          return c2
        lax.fori_loop(0, kmax, k_body, 0)
        return carry
      lax.fori_loop(0, (cnt + ROWC - 1) // ROWC, g_body, 0)

    pltpu.async_copy(src_hbm.at[pl.ds(0, B_E)], src_a, sem_s)
    pltpu.async_copy(dst_hbm.at[pl.ds(0, B_E)], dst_a, sem_d)

    def blk2_body(bb, carry):
      b0 = 2 * bb
      base0 = b0 * B_E
      base1 = base0 + B_E
      pltpu.make_async_copy(src_hbm.at[pl.ds(base0, B_E)], src_a,
                            sem_s).wait()
      pltpu.make_async_copy(dst_hbm.at[pl.ds(base0, B_E)], dst_a,
                            sem_d).wait()
      pltpu.async_copy(src_hbm.at[pl.ds(base1, B_E)], src_b, sem_s)
      pltpu.async_copy(dst_hbm.at[pl.ds(base1, B_E)], dst_b, sem_d)
      do_block(base0, src_a, dst_a)
      pltpu.make_async_copy(src_hbm.at[pl.ds(base1, B_E)], src_b,
                            sem_s).wait()
      pltpu.make_async_copy(dst_hbm.at[pl.ds(base1, B_E)], dst_b,
                            sem_d).wait()

      @pl.when(bb < NBLK // 2 - 1)
      def _():
        base2 = base1 + B_E
        pltpu.async_copy(src_hbm.at[pl.ds(base2, B_E)], src_a, sem_s)
        pltpu.async_copy(dst_hbm.at[pl.ds(base2, B_E)], dst_a, sem_d)
      do_block(base1, src_b, dst_b)
      return carry
    lax.fori_loop(0, NBLK // 2, blk2_body, 0)

    def fin(j, carry):
      invv = 1.0 / (zacc[j, :] + 1e-16)
      for f in range(d // 16):
        acc[j, pl.ds(f * 16, 16)] = acc[j, pl.ds(f * 16, 16)] * invv
      return carry
    lax.fori_loop(0, NLOC, fin, 0)

    pltpu.sync_copy(acc, out_hbm.at[pl.ds(lo, NLOC)])

  return k(src, dst, asrc, adst, h_pk)


def _pack_rows(h_perm):
  hb = h_perm.astype(jnp.bfloat16).reshape(N // 2, 128, 2)
  return jax.lax.bitcast_convert_type(hb, jnp.int32)


def _pack_alpha(asrc):
  ab = asrc.astype(jnp.bfloat16).reshape(N // 2, 2)
  return jax.lax.bitcast_convert_type(ab, jnp.int32)


def kernel(in_feat, g, W1, a_src1, a_dst1, W2, a_src2, a_dst2):
  src = g[0]
  dst = g[1]
  perm = jnp.asarray(PERM)

  W1p = W1[:, perm]
  a21 = jnp.stack([a_src1, a_dst1], axis=1)[perm]
  h1, al1 = _tc_proj(in_feat, W1p, a21, apply_relu=False)
  adst1_pad = jnp.pad(al1[:, 1], (0, N_PAD + 16 - N))
  out1 = _sc_layer(src, dst, _pack_alpha(al1[:, 0]), adst1_pad,
                   _pack_rows(h1), 128)[:N]

  d2 = W2.shape[1]
  W2p = jnp.pad(W2, ((0, 0), (0, 128 - d2)))[:, perm]
  a22 = jnp.pad(jnp.stack([a_src2, a_dst2], axis=1),
                ((0, 128 - d2), (0, 0)))[perm]
  h2, al2 = _tc_proj(out1, W2p, a22, apply_relu=True)
  adst2_pad = jnp.pad(al2[:, 1], (0, N_PAD + 16 - N))
  out2 = _sc_layer(src, dst, _pack_alpha(al2[:, 0]), adst2_pad,
                   _pack_rows(h2), d2)[:N]
  return out2


# speedup vs baseline: 1.0097x; 1.0097x over previous
import functools

import numpy as np

import jax
import jax.numpy as jnp
from jax import lax
from jax.experimental import pallas as pl
from jax.experimental.pallas import tpu as pltpu
from jax.experimental.pallas import tpu_sc as plsc

N = 10000
E = 320000
NEG_SLOPE = 0.2

NC = 2
NS = 16
NW = NC * NS
NLOC = 320
N_PAD = NW * NLOC
B_E = 400
NBLK = E // B_E
ROWC = 32
CH = 256
CAP = E + 16 * NBLK
SENT = 1 << 14

PERM = np.zeros(128, np.int32)
for _f in range(4):
  for _k in range(16):
    PERM[32 * _f + 2 * _k] = 32 * _f + _k
    PERM[32 * _f + 2 * _k + 1] = 32 * _f + 16 + _k


def _tc_proj(x, W, a2, apply_relu):
  n, _ = x.shape
  d_out = W.shape[1]

  def body(x_ref, w_ref, a_ref, h_ref, al_ref):
    xv = x_ref[...]
    if apply_relu:
      xv = jnp.maximum(xv, 0.0)
    h = jnp.dot(xv, w_ref[...], preferred_element_type=jnp.float32)
    h_ref[...] = h
    al_ref[...] = jnp.dot(h, a_ref[...], preferred_element_type=jnp.float32)

  return pl.pallas_call(
      body,
      out_shape=[
          jax.ShapeDtypeStruct((n, d_out), jnp.float32),
          jax.ShapeDtypeStruct((n, 2), jnp.float32),
      ],
  )(x, W, a2)


def _sc_layer(asrc, adst, h_pk, d, emit, src=None, dst=None,
              elist=None, counts=None):
  nfb = d // 32
  mesh = plsc.VectorSubcoreMesh(core_axis_name="c", subcore_axis_name="s")

  out_t = jax.ShapeDtypeStruct((N_PAD, d), jnp.float32)
  if emit:
    out_type = [out_t,
                jax.ShapeDtypeStruct((NW * CAP,), jnp.int32),
                jax.ShapeDtypeStruct((NW * 16,), jnp.int32)]
  else:
    out_type = out_t

  scratch = [
      pltpu.VMEM_SHARED((N // 2, 128), jnp.int32),
      pltpu.VMEM_SHARED((N // 2,), jnp.int32),
      pltpu.VMEM((NLOC + 16,), jnp.float32),
      pltpu.VMEM((ROWC,), jnp.int32),
      pltpu.VMEM((ROWC,), jnp.int32),
      pltpu.VMEM((ROWC,), jnp.float32),
      pltpu.VMEM((ROWC, 128), jnp.int32),
      pltpu.VMEM((NLOC, d), jnp.float32),
      pltpu.VMEM((NLOC, 16), jnp.float32),
      pltpu.VMEM((16,), jnp.int32),
      pltpu.SemaphoreType.DMA,
      pltpu.SemaphoreType.DMA,
      pltpu.SemaphoreType.DMA,
      pltpu.SemaphoreType.DMA,
      pltpu.SemaphoreType.DMA,
  ]
  if emit:
    scratch += [
        pltpu.VMEM((B_E,), jnp.int32),
        pltpu.VMEM((B_E,), jnp.int32),
        pltpu.VMEM((B_E,), jnp.int32),
        pltpu.VMEM((B_E,), jnp.int32),
        pltpu.VMEM((B_E + 16,), jnp.int32),
    ]
  else:
    scratch += [
        pltpu.VMEM((CH + 16,), jnp.int32),
        pltpu.VMEM((CH + 16,), jnp.int32),
    ]

  def body_common(w, lo, h_sh, asrc_sh, adst_loc, idx_rows, apair_chunk,
                  ew_chunk, rows, acc, zacc, sem, sem2):
    himask = jnp.full((16,), jnp.int32(-65536))

    def agg_chunk(sel_ref, off, navail):
      for q in range(ROWC // 16):
        sv = sel_ref[pl.ds(off + q * 16, 16)] & 32767
        idx_rows[pl.ds(q * 16, 16)] = jnp.minimum(sv >> 1, N // 2 - 1)
      pltpu.async_copy(h_sh.at[idx_rows], rows, sem)
      pltpu.async_copy(asrc_sh.at[idx_rows], apair_chunk, sem2)
      pltpu.make_async_copy(h_sh.at[idx_rows], rows, sem).wait()
      pltpu.make_async_copy(asrc_sh.at[idx_rows], apair_chunk, sem2).wait()

      for q in range(ROWC // 16):
        pk = sel_ref[pl.ds(off + q * 16, 16)]
        raw = apair_chunk[pl.ds(q * 16, 16)]
        asr = jnp.where((pk & 1) == 1,
                        plsc.bitcast(raw & himask, jnp.float32),
                        plsc.bitcast(raw << 16, jnp.float32))
        a = asr + plsc.load_gather(adst_loc,
                                   [jnp.minimum(pk >> 15, NLOC)])
        e = jnp.maximum(a, NEG_SLOPE * a)
        live = (pk & SENT) == 0
        ew_chunk[pl.ds(q * 16, 16)] = jnp.where(live, jnp.exp(e), 0.0)

      kmax = jnp.minimum(ROWC, navail)

      def k_body(ki, c2):
        s = sel_ref[pl.ds(off + ki, 16)][0]
        j = s >> 15
        colbase = (s & 1) * 64
        sw = ew_chunk[pl.ds(ki, 16)][0]
        for f in range(nfb):
          raw = rows[ki, pl.ds(colbase + f * 16, 16)]
          evn = plsc.bitcast(raw << 16, jnp.float32)
          odd = plsc.bitcast(raw & himask, jnp.float32)
          acc[j, pl.ds(f * 32, 16)] = (
              acc[j, pl.ds(f * 32, 16)] + evn * sw)
          acc[j, pl.ds(f * 32 + 16, 16)] = (
              acc[j, pl.ds(f * 32 + 16, 16)] + odd * sw)
        zacc[j, :] = zacc[j, :] + sw
        return c2
      lax.fori_loop(0, kmax, k_body, 0)
    return agg_chunk

  if emit:
    @functools.partial(
        pl.kernel, out_type=out_type, mesh=mesh,
        compiler_params=pltpu.CompilerParams(needs_layout_passes=False),
        scratch_types=scratch)
    def k(src_hbm, dst_hbm, asrc_hbm, adst_hbm, hpk_hbm,
          out_hbm, elist_hbm, counts_hbm,
          h_sh, asrc_sh, adst_loc, idx_rows, apair_chunk, ew_chunk,
          rows, acc, zacc, cntb, sem, sem2, sem_s, sem_d, sem_w,
          src_a, dst_a, src_b, dst_b, sel_pk):
      s_id = lax.axis_index("s")
      w = s_id * NC + lax.axis_index("c")
      lo = w * NLOC

      @pl.when(s_id < 5)
      def _():
        pltpu.sync_copy(hpk_hbm.at[pl.ds(s_id * 1000, 1000)],
                        h_sh.at[pl.ds(s_id * 1000, 1000)])

      @pl.when(s_id == 5)
      def _():
        pltpu.sync_copy(asrc_hbm, asrc_sh)

      pltpu.sync_copy(adst_hbm.at[pl.ds(lo, NLOC + 16)], adst_loc)
      plsc.subcore_barrier()

      zeros16f = jnp.zeros((16,), jnp.float32)
      zeros16i = jnp.zeros((16,), jnp.int32)
      sent16 = jnp.full((16,), jnp.int32(SENT))

      def zrow(j, carry):
        for f in range(d // 16):
          acc[j, pl.ds(f * 16, 16)] = zeros16f
        zacc[j, :] = zeros16f
        return carry
      lax.fori_loop(0, NLOC, zrow, 0)

      def zsel(i, carry):
        sel_pk[pl.ds(i * 16, 16)] = zeros16i
        return carry
      lax.fori_loop(0, (B_E + 16) // 16, zsel, 0)

      agg_chunk = body_common(w, lo, h_sh, asrc_sh, adst_loc, idx_rows,
                              apair_chunk, ew_chunk, rows, acc, zacc,
                              sem, sem2)

      def do_block(src_v, dst_v, cursor):
        def sel_body(i, cur):
          sv = src_v[pl.ds(i * 16, 16)]
          dv = dst_v[pl.ds(i * 16, 16)]
          m = (dv >= lo) & (dv < lo + NLOC)
          c = jnp.cumsum(m.astype(jnp.int32))
          pos = jnp.where(m, cur + c - 1, B_E + 8)
          plsc.store_scatter(sel_pk, [pos], ((dv - lo) << 15) | sv)
          return cur + c[15]
        cnt = lax.fori_loop(0, B_E // 16, sel_body, jnp.int32(0))
        sel_pk[pl.ds(cnt, 16)] = sent16
        nch = (cnt + 15) // 16

        def wr(ci, carry):
          pltpu.async_copy(
              sel_pk.at[pl.ds(ci * 16, 16)],
              elist_hbm.at[pl.ds(pl.multiple_of(w * CAP + cursor + ci * 16, 8), 16)], sem_w)
          return carry
        lax.fori_loop(0, nch, wr, 0)

        def g_body(gi, carry):
          agg_chunk(sel_pk, gi * ROWC, cnt - gi * ROWC)
          return carry
        lax.fori_loop(0, (cnt + ROWC - 1) // ROWC, g_body, 0)

        def wrw(ci, carry):
          pltpu.make_async_copy(
              sel_pk.at[pl.ds(0, 16)],
              elist_hbm.at[pl.ds(w * CAP, 16)], sem_w).wait()
          return carry
        lax.fori_loop(0, nch, wrw, 0)
        return cursor + nch * 16

      pltpu.async_copy(src_hbm.at[pl.ds(0, B_E)], src_a, sem_s)
      pltpu.async_copy(dst_hbm.at[pl.ds(0, B_E)], dst_a, sem_d)

      def blk2_body(bb, cursor):
        base0 = 2 * bb * B_E
        base1 = base0 + B_E
        pltpu.make_async_copy(src_hbm.at[pl.ds(base0, B_E)], src_a,
                              sem_s).wait()
        pltpu.make_async_copy(dst_hbm.at[pl.ds(base0, B_E)], dst_a,
                              sem_d).wait()
        pltpu.async_copy(src_hbm.at[pl.ds(base1, B_E)], src_b, sem_s)
        pltpu.async_copy(dst_hbm.at[pl.ds(base1, B_E)], dst_b, sem_d)
        cursor = do_block(src_a, dst_a, cursor)
        pltpu.make_async_copy(src_hbm.at[pl.ds(base1, B_E)], src_b,
                              sem_s).wait()
        pltpu.make_async_copy(dst_hbm.at[pl.ds(base1, B_E)], dst_b,
                              sem_d).wait()

        @pl.when(bb < NBLK // 2 - 1)
        def _():
          base2 = base1 + B_E
          pltpu.async_copy(src_hbm.at[pl.ds(base2, B_E)], src_a, sem_s)
          pltpu.async_copy(dst_hbm.at[pl.ds(base2, B_E)], dst_a, sem_d)
        cursor = do_block(src_b, dst_b, cursor)
        return cursor
      total = lax.fori_loop(0, NBLK // 2, blk2_body, jnp.int32(0))

      cntb[...] = jnp.full((16,), total)
      pltpu.sync_copy(cntb, counts_hbm.at[pl.ds(w * 16, 16)])

      def fin(j, carry):
        invv = 1.0 / (zacc[j, :] + 1e-16)
        for f in range(d // 16):
          acc[j, pl.ds(f * 16, 16)] = acc[j, pl.ds(f * 16, 16)] * invv
        return carry
      lax.fori_loop(0, NLOC, fin, 0)
      pltpu.sync_copy(acc.at[pl.ds(0, NLOC)], out_hbm.at[pl.ds(lo, NLOC)])

    return k(src, dst, asrc, adst, h_pk)

  else:
    @functools.partial(
        pl.kernel, out_type=out_type, mesh=mesh,
        compiler_params=pltpu.CompilerParams(needs_layout_passes=False),
        scratch_types=scratch)
    def k(elist_hbm, counts_hbm, asrc_hbm, adst_hbm, hpk_hbm, out_hbm,
          h_sh, asrc_sh, adst_loc, idx_rows, apair_chunk, ew_chunk,
          rows, acc, zacc, cntb, sem, sem2, sem_s, sem_d, sem_w,
          sel_a, sel_b):
      s_id = lax.axis_index("s")
      w = s_id * NC + lax.axis_index("c")
      lo = w * NLOC

      @pl.when(s_id < 5)
      def _():
        pltpu.sync_copy(hpk_hbm.at[pl.ds(s_id * 1000, 1000)],
                        h_sh.at[pl.ds(s_id * 1000, 1000)])

      @pl.when(s_id == 5)
      def _():
        pltpu.sync_copy(asrc_hbm, asrc_sh)

      pltpu.sync_copy(adst_hbm.at[pl.ds(lo, NLOC + 16)], adst_loc)
      pltpu.sync_copy(counts_hbm.at[pl.ds(w * 16, 16)], cntb)
      plsc.subcore_barrier()

      total = cntb[pl.ds(0, 16)][0]

      zeros16f = jnp.zeros((16,), jnp.float32)

      def zrow(j, carry):
        for f in range(d // 16):
          acc[j, pl.ds(f * 16, 16)] = zeros16f
        zacc[j, :] = zeros16f
        return carry
      lax.fori_loop(0, NLOC, zrow, 0)

      agg_chunk = body_common(w, lo, h_sh, asrc_sh, adst_loc, idx_rows,
                              apair_chunk, ew_chunk, rows, acc, zacc,
                              sem, sem2)

      def do_chunk(sel_ref, base):
        for gi in range(CH // ROWC):
          agg_chunk(sel_ref, gi * ROWC, total - base - gi * ROWC)

      pltpu.async_copy(elist_hbm.at[pl.ds(w * CAP, CH)], sel_a.at[pl.ds(0, CH)], sem_s)
      npair = (total + 2 * CH - 1) // (2 * CH)

      def ch2_body(cc, carry):
        base0 = 2 * cc * CH
        base1 = base0 + CH
        pltpu.make_async_copy(elist_hbm.at[pl.ds(w * CAP, CH)], sel_a.at[pl.ds(0, CH)],
                              sem_s).wait()
        b1c = jnp.minimum(base1, CAP - CH)
        pltpu.async_copy(elist_hbm.at[pl.ds(pl.multiple_of(w * CAP + b1c, 8), CH)], sel_b.at[pl.ds(0, CH)], sem_d)
        do_chunk(sel_a, base0)
        pltpu.make_async_copy(elist_hbm.at[pl.ds(w * CAP, CH)], sel_b.at[pl.ds(0, CH)],
                              sem_d).wait()

        @pl.when(cc < npair - 1)
        def _():
          base2 = jnp.minimum(base1 + CH, CAP - CH)
          pltpu.async_copy(elist_hbm.at[pl.ds(pl.multiple_of(w * CAP + base2, 8), CH)],
                           sel_a.at[pl.ds(0, CH)], sem_s)
        do_chunk(sel_b, base1)
        return carry
      lax.fori_loop(0, npair, ch2_body, 0)

      def fin(j, carry):
        invv = 1.0 / (zacc[j, :] + 1e-16)
        for f in range(d // 16):
          acc[j, pl.ds(f * 16, 16)] = acc[j, pl.ds(f * 16, 16)] * invv
        return carry
      lax.fori_loop(0, NLOC, fin, 0)
      pltpu.sync_copy(acc.at[pl.ds(0, NLOC)], out_hbm.at[pl.ds(lo, NLOC)])

    return k(elist, counts, asrc, adst, h_pk)


def _pack_rows(h_perm):
  hb = h_perm.astype(jnp.bfloat16).reshape(N // 2, 128, 2)
  return jax.lax.bitcast_convert_type(hb, jnp.int32)


def _pack_alpha(asrc):
  ab = asrc.astype(jnp.bfloat16).reshape(N // 2, 2)
  return jax.lax.bitcast_convert_type(ab, jnp.int32)


def kernel(in_feat, g, W1, a_src1, a_dst1, W2, a_src2, a_dst2):
  src = g[0]
  dst = g[1]
  perm = jnp.asarray(PERM)

  W1p = W1[:, perm]
  a21 = jnp.stack([a_src1, a_dst1], axis=1)[perm]
  h1, al1 = _tc_proj(in_feat, W1p, a21, apply_relu=False)
  adst1_pad = jnp.pad(al1[:, 1], (0, N_PAD + 16 - N))
  out1p, elist, counts = _sc_layer(_pack_alpha(al1[:, 0]), adst1_pad,
                                   _pack_rows(h1), 128, emit=True,
                                   src=src, dst=dst)
  out1 = out1p[:N]

  d2 = W2.shape[1]
  W2p = jnp.pad(W2, ((0, 0), (0, 128 - d2)))[:, perm]
  a22 = jnp.pad(jnp.stack([a_src2, a_dst2], axis=1),
                ((0, 128 - d2), (0, 0)))[perm]
  h2, al2 = _tc_proj(out1, W2p, a22, apply_relu=True)
  adst2_pad = jnp.pad(al2[:, 1], (0, N_PAD + 16 - N))
  out2 = _sc_layer(_pack_alpha(al2[:, 0]), adst2_pad, _pack_rows(h2),
                   d2, emit=False, elist=elist, counts=counts)[:N]
  return out2


# speedup vs baseline: 1.0570x; 1.0469x over previous
import functools

import numpy as np

import jax
import jax.numpy as jnp
from jax import lax
from jax.experimental import pallas as pl
from jax.experimental.pallas import tpu as pltpu
from jax.experimental.pallas import tpu_sc as plsc

N = 10000
E = 320000
NEG_SLOPE = 0.2

NC = 2
NS = 16
NW = NC * NS
NLOC = 320
N_PAD = NW * NLOC
B_E = 640
NBLK = E // B_E
ROWC = 32
CH = 256
CAP = E + 16 * NBLK
SENT = 1 << 14

PERM = np.zeros(128, np.int32)
for _f in range(4):
  for _k in range(16):
    PERM[32 * _f + 2 * _k] = 32 * _f + _k
    PERM[32 * _f + 2 * _k + 1] = 32 * _f + 16 + _k


def _tc_proj(x, W, a2, apply_relu):
  n, _ = x.shape
  d_out = W.shape[1]

  def body(x_ref, w_ref, a_ref, h_ref, al_ref):
    xv = x_ref[...]
    if apply_relu:
      xv = jnp.maximum(xv, 0.0)
    h = jnp.dot(xv, w_ref[...], preferred_element_type=jnp.float32)
    h_ref[...] = h
    al_ref[...] = jnp.dot(h, a_ref[...], preferred_element_type=jnp.float32)

  return pl.pallas_call(
      body,
      out_shape=[
          jax.ShapeDtypeStruct((n, d_out), jnp.float32),
          jax.ShapeDtypeStruct((n, 2), jnp.float32),
      ],
  )(x, W, a2)


def _sc_layer(asrc, adst, h_pk, d, emit, src=None, dst=None,
              elist=None, counts=None):
  nfb = d // 32
  mesh = plsc.VectorSubcoreMesh(core_axis_name="c", subcore_axis_name="s")

  out_t = jax.ShapeDtypeStruct((N_PAD, d), jnp.float32)
  if emit:
    out_type = [out_t,
                jax.ShapeDtypeStruct((NW * CAP,), jnp.int32),
                jax.ShapeDtypeStruct((NW * 16,), jnp.int32)]
  else:
    out_type = out_t

  scratch = [
      pltpu.VMEM_SHARED((N // 2, 128), jnp.int32),
      pltpu.VMEM_SHARED((N // 2,), jnp.int32),
      pltpu.VMEM((NLOC + 16,), jnp.float32),
      pltpu.VMEM((ROWC,), jnp.int32),
      pltpu.VMEM((ROWC,), jnp.int32),
      pltpu.VMEM((ROWC,), jnp.float32),
      pltpu.VMEM((ROWC, 128), jnp.int32),
      pltpu.VMEM((NLOC, d), jnp.float32),
      pltpu.VMEM((NLOC, 16), jnp.float32),
      pltpu.VMEM((16,), jnp.int32),
      pltpu.SemaphoreType.DMA,
      pltpu.SemaphoreType.DMA,
      pltpu.SemaphoreType.DMA,
      pltpu.SemaphoreType.DMA,
      pltpu.SemaphoreType.DMA,
  ]
  if emit:
    scratch += [
        pltpu.VMEM((B_E,), jnp.int32),
        pltpu.VMEM((B_E,), jnp.int32),
        pltpu.VMEM((B_E,), jnp.int32),
        pltpu.VMEM((B_E,), jnp.int32),
        pltpu.VMEM((B_E + 16,), jnp.int32),
    ]
  else:
    scratch += [
        pltpu.VMEM((CH + 16,), jnp.int32),
        pltpu.VMEM((CH + 16,), jnp.int32),
    ]

  def body_common(w, lo, h_sh, asrc_sh, adst_loc, idx_rows, apair_chunk,
                  ew_chunk, rows, acc, zacc, sem, sem2):
    himask = jnp.full((16,), jnp.int32(-65536))

    def agg_chunk(sel_ref, off, navail):
      for q in range(ROWC // 16):
        sv = sel_ref[pl.ds(off + q * 16, 16)] & 32767
        idx_rows[pl.ds(q * 16, 16)] = jnp.minimum(sv >> 1, N // 2 - 1)
      pltpu.async_copy(h_sh.at[idx_rows], rows, sem)
      pltpu.async_copy(asrc_sh.at[idx_rows], apair_chunk, sem2)
      pltpu.make_async_copy(h_sh.at[idx_rows], rows, sem).wait()
      pltpu.make_async_copy(asrc_sh.at[idx_rows], apair_chunk, sem2).wait()

      for q in range(ROWC // 16):
        pk = sel_ref[pl.ds(off + q * 16, 16)]
        raw = apair_chunk[pl.ds(q * 16, 16)]
        asr = jnp.where((pk & 1) == 1,
                        plsc.bitcast(raw & himask, jnp.float32),
                        plsc.bitcast(raw << 16, jnp.float32))
        a = asr + plsc.load_gather(adst_loc,
                                   [jnp.minimum(pk >> 15, NLOC)])
        e = jnp.maximum(a, NEG_SLOPE * a)
        live = (pk & SENT) == 0
        ew_chunk[pl.ds(q * 16, 16)] = jnp.where(live, jnp.exp(e), 0.0)

      kmax = jnp.minimum(ROWC, navail)

      def k_body(ki, c2):
        s = sel_ref[pl.ds(off + ki, 16)][0]
        j = s >> 15
        colbase = (s & 1) * 64
        sw = ew_chunk[pl.ds(ki, 16)][0]
        for f in range(nfb):
          raw = rows[ki, pl.ds(colbase + f * 16, 16)]
          evn = plsc.bitcast(raw << 16, jnp.float32)
          odd = plsc.bitcast(raw & himask, jnp.float32)
          acc[j, pl.ds(f * 32, 16)] = (
              acc[j, pl.ds(f * 32, 16)] + evn * sw)
          acc[j, pl.ds(f * 32 + 16, 16)] = (
              acc[j, pl.ds(f * 32 + 16, 16)] + odd * sw)
        zacc[j, :] = zacc[j, :] + sw
        return c2
      lax.fori_loop(0, kmax, k_body, 0)
    return agg_chunk

  if emit:
    @functools.partial(
        pl.kernel, out_type=out_type, mesh=mesh,
        compiler_params=pltpu.CompilerParams(needs_layout_passes=False),
        scratch_types=scratch)
    def k(src_hbm, dst_hbm, asrc_hbm, adst_hbm, hpk_hbm,
          out_hbm, elist_hbm, counts_hbm,
          h_sh, asrc_sh, adst_loc, idx_rows, apair_chunk, ew_chunk,
          rows, acc, zacc, cntb, sem, sem2, sem_s, sem_d, sem_w,
          src_a, dst_a, src_b, dst_b, sel_pk):
      s_id = lax.axis_index("s")
      w = s_id * NC + lax.axis_index("c")
      lo = w * NLOC

      @pl.when(s_id < 5)
      def _():
        pltpu.sync_copy(hpk_hbm.at[pl.ds(s_id * 1000, 1000)],
                        h_sh.at[pl.ds(s_id * 1000, 1000)])

      @pl.when(s_id == 5)
      def _():
        pltpu.sync_copy(asrc_hbm, asrc_sh)

      pltpu.sync_copy(adst_hbm.at[pl.ds(lo, NLOC + 16)], adst_loc)
      plsc.subcore_barrier()

      zeros16f = jnp.zeros((16,), jnp.float32)
      zeros16i = jnp.zeros((16,), jnp.int32)
      sent16 = jnp.full((16,), jnp.int32(SENT))

      def zrow(j, carry):
        for f in range(d // 16):
          acc[j, pl.ds(f * 16, 16)] = zeros16f
        zacc[j, :] = zeros16f
        return carry
      lax.fori_loop(0, NLOC, zrow, 0)

      def zsel(i, carry):
        sel_pk[pl.ds(i * 16, 16)] = zeros16i
        return carry
      lax.fori_loop(0, (B_E + 16) // 16, zsel, 0)

      agg_chunk = body_common(w, lo, h_sh, asrc_sh, adst_loc, idx_rows,
                              apair_chunk, ew_chunk, rows, acc, zacc,
                              sem, sem2)

      def do_block(src_v, dst_v, cursor):
        def sel_body(i, cur):
          sv = src_v[pl.ds(i * 16, 16)]
          dv = dst_v[pl.ds(i * 16, 16)]
          m = (dv >= lo) & (dv < lo + NLOC)
          c = jnp.cumsum(m.astype(jnp.int32))
          pos = jnp.where(m, cur + c - 1, B_E + 8)
          plsc.store_scatter(sel_pk, [pos], ((dv - lo) << 15) | sv)
          return cur + c[15]
        cnt = lax.fori_loop(0, B_E // 16, sel_body, jnp.int32(0))
        sel_pk[pl.ds(cnt, 16)] = sent16
        nch = (cnt + 15) // 16

        def wr(ci, carry):
          pltpu.async_copy(
              sel_pk.at[pl.ds(ci * 16, 16)],
              elist_hbm.at[pl.ds(pl.multiple_of(w * CAP + cursor + ci * 16, 8), 16)], sem_w)
          return carry
        lax.fori_loop(0, nch, wr, 0)

        def g_body(gi, carry):
          agg_chunk(sel_pk, gi * ROWC, cnt - gi * ROWC)
          return carry
        lax.fori_loop(0, (cnt + ROWC - 1) // ROWC, g_body, 0)

        def wrw(ci, carry):
          pltpu.make_async_copy(
              sel_pk.at[pl.ds(0, 16)],
              elist_hbm.at[pl.ds(w * CAP, 16)], sem_w).wait()
          return carry
        lax.fori_loop(0, nch, wrw, 0)
        return cursor + nch * 16

      pltpu.async_copy(src_hbm.at[pl.ds(0, B_E)], src_a, sem_s)
      pltpu.async_copy(dst_hbm.at[pl.ds(0, B_E)], dst_a, sem_d)

      def blk2_body(bb, cursor):
        base0 = 2 * bb * B_E
        base1 = base0 + B_E
        pltpu.make_async_copy(src_hbm.at[pl.ds(base0, B_E)], src_a,
                              sem_s).wait()
        pltpu.make_async_copy(dst_hbm.at[pl.ds(base0, B_E)], dst_a,
                              sem_d).wait()
        pltpu.async_copy(src_hbm.at[pl.ds(base1, B_E)], src_b, sem_s)
        pltpu.async_copy(dst_hbm.at[pl.ds(base1, B_E)], dst_b, sem_d)
        cursor = do_block(src_a, dst_a, cursor)
        pltpu.make_async_copy(src_hbm.at[pl.ds(base1, B_E)], src_b,
                              sem_s).wait()
        pltpu.make_async_copy(dst_hbm.at[pl.ds(base1, B_E)], dst_b,
                              sem_d).wait()

        @pl.when(bb < NBLK // 2 - 1)
        def _():
          base2 = base1 + B_E
          pltpu.async_copy(src_hbm.at[pl.ds(base2, B_E)], src_a, sem_s)
          pltpu.async_copy(dst_hbm.at[pl.ds(base2, B_E)], dst_a, sem_d)
        cursor = do_block(src_b, dst_b, cursor)
        return cursor
      total = lax.fori_loop(0, NBLK // 2, blk2_body, jnp.int32(0))

      cntb[...] = jnp.full((16,), total)
      pltpu.sync_copy(cntb, counts_hbm.at[pl.ds(w * 16, 16)])

      def fin(j, carry):
        invv = 1.0 / (zacc[j, :] + 1e-16)
        for f in range(d // 16):
          acc[j, pl.ds(f * 16, 16)] = acc[j, pl.ds(f * 16, 16)] * invv
        return carry
      lax.fori_loop(0, NLOC, fin, 0)
      pltpu.sync_copy(acc.at[pl.ds(0, NLOC)], out_hbm.at[pl.ds(lo, NLOC)])

    return k(src, dst, asrc, adst, h_pk)

  else:
    @functools.partial(
        pl.kernel, out_type=out_type, mesh=mesh,
        compiler_params=pltpu.CompilerParams(needs_layout_passes=False),
        scratch_types=scratch)
    def k(elist_hbm, counts_hbm, asrc_hbm, adst_hbm, hpk_hbm, out_hbm,
          h_sh, asrc_sh, adst_loc, idx_rows, apair_chunk, ew_chunk,
          rows, acc, zacc, cntb, sem, sem2, sem_s, sem_d, sem_w,
          sel_a, sel_b):
      s_id = lax.axis_index("s")
      w = s_id * NC + lax.axis_index("c")
      lo = w * NLOC

      @pl.when(s_id < 5)
      def _():
        pltpu.sync_copy(hpk_hbm.at[pl.ds(s_id * 1000, 1000)],
                        h_sh.at[pl.ds(s_id * 1000, 1000)])

      @pl.when(s_id == 5)
      def _():
        pltpu.sync_copy(asrc_hbm, asrc_sh)

      pltpu.sync_copy(adst_hbm.at[pl.ds(lo, NLOC + 16)], adst_loc)
      pltpu.sync_copy(counts_hbm.at[pl.ds(w * 16, 16)], cntb)
      plsc.subcore_barrier()

      total = cntb[pl.ds(0, 16)][0]

      zeros16f = jnp.zeros((16,), jnp.float32)

      def zrow(j, carry):
        for f in range(d // 16):
          acc[j, pl.ds(f * 16, 16)] = zeros16f
        zacc[j, :] = zeros16f
        return carry
      lax.fori_loop(0, NLOC, zrow, 0)

      agg_chunk = body_common(w, lo, h_sh, asrc_sh, adst_loc, idx_rows,
                              apair_chunk, ew_chunk, rows, acc, zacc,
                              sem, sem2)

      def do_chunk(sel_ref, base):
        for gi in range(CH // ROWC):
          agg_chunk(sel_ref, gi * ROWC, total - base - gi * ROWC)

      pltpu.async_copy(elist_hbm.at[pl.ds(w * CAP, CH)], sel_a.at[pl.ds(0, CH)], sem_s)
      npair = (total + 2 * CH - 1) // (2 * CH)

      def ch2_body(cc, carry):
        base0 = 2 * cc * CH
        base1 = base0 + CH
        pltpu.make_async_copy(elist_hbm.at[pl.ds(w * CAP, CH)], sel_a.at[pl.ds(0, CH)],
                              sem_s).wait()
        b1c = jnp.minimum(base1, CAP - CH)
        pltpu.async_copy(elist_hbm.at[pl.ds(pl.multiple_of(w * CAP + b1c, 8), CH)], sel_b.at[pl.ds(0, CH)], sem_d)
        do_chunk(sel_a, base0)
        pltpu.make_async_copy(elist_hbm.at[pl.ds(w * CAP, CH)], sel_b.at[pl.ds(0, CH)],
                              sem_d).wait()

        @pl.when(cc < npair - 1)
        def _():
          base2 = jnp.minimum(base1 + CH, CAP - CH)
          pltpu.async_copy(elist_hbm.at[pl.ds(pl.multiple_of(w * CAP + base2, 8), CH)],
                           sel_a.at[pl.ds(0, CH)], sem_s)
        do_chunk(sel_b, base1)
        return carry
      lax.fori_loop(0, npair, ch2_body, 0)

      def fin(j, carry):
        invv = 1.0 / (zacc[j, :] + 1e-16)
        for f in range(d // 16):
          acc[j, pl.ds(f * 16, 16)] = acc[j, pl.ds(f * 16, 16)] * invv
        return carry
      lax.fori_loop(0, NLOC, fin, 0)
      pltpu.sync_copy(acc.at[pl.ds(0, NLOC)], out_hbm.at[pl.ds(lo, NLOC)])

    return k(elist, counts, asrc, adst, h_pk)


def _pack_rows(h_perm):
  hb = h_perm.astype(jnp.bfloat16).reshape(N // 2, 128, 2)
  return jax.lax.bitcast_convert_type(hb, jnp.int32)


def _pack_alpha(asrc):
  ab = asrc.astype(jnp.bfloat16).reshape(N // 2, 2)
  return jax.lax.bitcast_convert_type(ab, jnp.int32)


def kernel(in_feat, g, W1, a_src1, a_dst1, W2, a_src2, a_dst2):
  src = g[0]
  dst = g[1]
  perm = jnp.asarray(PERM)

  W1p = W1[:, perm]
  a21 = jnp.stack([a_src1, a_dst1], axis=1)[perm]
  h1, al1 = _tc_proj(in_feat, W1p, a21, apply_relu=False)
  adst1_pad = jnp.pad(al1[:, 1], (0, N_PAD + 16 - N))
  out1p, elist, counts = _sc_layer(_pack_alpha(al1[:, 0]), adst1_pad,
                                   _pack_rows(h1), 128, emit=True,
                                   src=src, dst=dst)
  out1 = out1p[:N]

  d2 = W2.shape[1]
  W2p = jnp.pad(W2, ((0, 0), (0, 128 - d2)))[:, perm]
  a22 = jnp.pad(jnp.stack([a_src2, a_dst2], axis=1),
                ((0, 128 - d2), (0, 0)))[perm]
  h2, al2 = _tc_proj(out1, W2p, a22, apply_relu=True)
  adst2_pad = jnp.pad(al2[:, 1], (0, N_PAD + 16 - N))
  out2 = _sc_layer(_pack_alpha(al2[:, 0]), adst2_pad, _pack_rows(h2),
                   d2, emit=False, elist=elist, counts=counts)[:N]
  return out2


# speedup vs baseline: 1.0980x; 1.0387x over previous
import functools

import numpy as np

import jax
import jax.numpy as jnp
from jax import lax
from jax.experimental import pallas as pl
from jax.experimental.pallas import tpu as pltpu
from jax.experimental.pallas import tpu_sc as plsc

N = 10000
E = 320000
NEG_SLOPE = 0.2

NC = 2
NS = 16
NW = NC * NS
NLOC = 320
N_PAD = NW * NLOC
B_E = 640
NBLK = E // B_E
ROWC = 32
CH = 256
CAP = E + 16 * NBLK
SENT = 1 << 14

PERM = np.zeros(128, np.int32)
for _f in range(4):
  for _k in range(16):
    PERM[32 * _f + 2 * _k] = 32 * _f + _k
    PERM[32 * _f + 2 * _k + 1] = 32 * _f + 16 + _k


def _tc_proj(x, W, a2, apply_relu):
  n, _ = x.shape
  d_out = W.shape[1]

  def body(x_ref, w_ref, a_ref, h_ref, al_ref):
    xv = x_ref[...]
    if apply_relu:
      xv = jnp.maximum(xv, 0.0)
    h = jnp.dot(xv, w_ref[...], preferred_element_type=jnp.float32)
    h_ref[...] = h
    al_ref[...] = jnp.dot(h, a_ref[...], preferred_element_type=jnp.float32)

  return pl.pallas_call(
      body,
      out_shape=[
          jax.ShapeDtypeStruct((n, d_out), jnp.float32),
          jax.ShapeDtypeStruct((n, 2), jnp.float32),
      ],
  )(x, W, a2)


def _sc_layer(asrc, adst, h_pk, d, emit, src=None, dst=None,
              elist=None, counts=None):
  nfb = d // 32
  rowc = ROWC if emit else 16
  mesh = plsc.VectorSubcoreMesh(core_axis_name="c", subcore_axis_name="s")

  out_t = jax.ShapeDtypeStruct((N_PAD, d), jnp.float32)
  if emit:
    out_type = [out_t,
                jax.ShapeDtypeStruct((NW * CAP,), jnp.int32),
                jax.ShapeDtypeStruct((NW * 16,), jnp.int32)]
  else:
    out_type = out_t

  scratch = [
      pltpu.VMEM_SHARED((N // 2, 128), jnp.int32),
      pltpu.VMEM_SHARED((N // 2,), jnp.int32),
      pltpu.VMEM((NLOC + 16,), jnp.float32),
      pltpu.VMEM((rowc,), jnp.int32),
      pltpu.VMEM((rowc,), jnp.int32),
      pltpu.VMEM((rowc,), jnp.float32),
      pltpu.VMEM((rowc, 128), jnp.int32),
      pltpu.VMEM((NLOC, d), jnp.float32),
      pltpu.VMEM((NLOC, 16), jnp.float32),
      pltpu.VMEM((16,), jnp.int32),
      pltpu.SemaphoreType.DMA,
      pltpu.SemaphoreType.DMA,
      pltpu.SemaphoreType.DMA,
      pltpu.SemaphoreType.DMA,
      pltpu.SemaphoreType.DMA,
  ]
  if emit:
    scratch += [
        pltpu.VMEM((B_E,), jnp.int32),
        pltpu.VMEM((B_E,), jnp.int32),
        pltpu.VMEM((B_E,), jnp.int32),
        pltpu.VMEM((B_E,), jnp.int32),
        pltpu.VMEM((B_E + 16,), jnp.int32),
    ]
  else:
    scratch += [
        pltpu.VMEM((CH + 16,), jnp.int32),
        pltpu.VMEM((CH + 16,), jnp.int32),
        pltpu.VMEM((rowc,), jnp.int32),
        pltpu.VMEM((rowc,), jnp.int32),
        pltpu.VMEM((rowc, 128), jnp.int32),
        pltpu.SemaphoreType.DMA,
        pltpu.SemaphoreType.DMA,
    ]

  def body_common(w, lo, h_sh, asrc_sh, adst_loc, idx_rows, apair_chunk,
                  ew_chunk, rows, acc, zacc, sem, sem2):
    himask = jnp.full((16,), jnp.int32(-65536))

    def agg_chunk(sel_ref, off, navail):
      for q in range(ROWC // 16):
        sv = sel_ref[pl.ds(off + q * 16, 16)] & 32767
        idx_rows[pl.ds(q * 16, 16)] = jnp.minimum(sv >> 1, N // 2 - 1)
      pltpu.async_copy(h_sh.at[idx_rows], rows, sem)
      pltpu.async_copy(asrc_sh.at[idx_rows], apair_chunk, sem2)
      pltpu.make_async_copy(h_sh.at[idx_rows], rows, sem).wait()
      pltpu.make_async_copy(asrc_sh.at[idx_rows], apair_chunk, sem2).wait()

      for q in range(ROWC // 16):
        pk = sel_ref[pl.ds(off + q * 16, 16)]
        raw = apair_chunk[pl.ds(q * 16, 16)]
        asr = jnp.where((pk & 1) == 1,
                        plsc.bitcast(raw & himask, jnp.float32),
                        plsc.bitcast(raw << 16, jnp.float32))
        a = asr + plsc.load_gather(adst_loc,
                                   [jnp.minimum(pk >> 15, NLOC)])
        e = jnp.maximum(a, NEG_SLOPE * a)
        live = (pk & SENT) == 0
        ew_chunk[pl.ds(q * 16, 16)] = jnp.where(live, jnp.exp(e), 0.0)

      kmax = jnp.minimum(ROWC, navail)

      def k_body(ki, c2):
        s = sel_ref[pl.ds(off + ki, 16)][0]
        j = s >> 15
        colbase = (s & 1) * 64
        sw = ew_chunk[pl.ds(ki, 16)][0]
        for f in range(nfb):
          raw = rows[ki, pl.ds(colbase + f * 16, 16)]
          evn = plsc.bitcast(raw << 16, jnp.float32)
          odd = plsc.bitcast(raw & himask, jnp.float32)
          acc[j, pl.ds(f * 32, 16)] = (
              acc[j, pl.ds(f * 32, 16)] + evn * sw)
          acc[j, pl.ds(f * 32 + 16, 16)] = (
              acc[j, pl.ds(f * 32 + 16, 16)] + odd * sw)
        zacc[j, :] = zacc[j, :] + sw
        return c2
      lax.fori_loop(0, kmax, k_body, 0)
    return agg_chunk

  if emit:
    @functools.partial(
        pl.kernel, out_type=out_type, mesh=mesh,
        compiler_params=pltpu.CompilerParams(needs_layout_passes=False),
        scratch_types=scratch)
    def k(src_hbm, dst_hbm, asrc_hbm, adst_hbm, hpk_hbm,
          out_hbm, elist_hbm, counts_hbm,
          h_sh, asrc_sh, adst_loc, idx_rows, apair_chunk, ew_chunk,
          rows, acc, zacc, cntb, sem, sem2, sem_s, sem_d, sem_w,
          src_a, dst_a, src_b, dst_b, sel_pk):
      s_id = lax.axis_index("s")
      w = s_id * NC + lax.axis_index("c")
      lo = w * NLOC

      @pl.when(s_id < 5)
      def _():
        pltpu.sync_copy(hpk_hbm.at[pl.ds(s_id * 1000, 1000)],
                        h_sh.at[pl.ds(s_id * 1000, 1000)])

      @pl.when(s_id == 5)
      def _():
        pltpu.sync_copy(asrc_hbm, asrc_sh)

      pltpu.sync_copy(adst_hbm.at[pl.ds(lo, NLOC + 16)], adst_loc)
      plsc.subcore_barrier()

      zeros16f = jnp.zeros((16,), jnp.float32)
      zeros16i = jnp.zeros((16,), jnp.int32)
      sent16 = jnp.full((16,), jnp.int32(SENT))

      def zrow(j, carry):
        for f in range(d // 16):
          acc[j, pl.ds(f * 16, 16)] = zeros16f
        zacc[j, :] = zeros16f
        return carry
      lax.fori_loop(0, NLOC, zrow, 0)

      def zsel(i, carry):
        sel_pk[pl.ds(i * 16, 16)] = zeros16i
        return carry
      lax.fori_loop(0, (B_E + 16) // 16, zsel, 0)

      agg_chunk = body_common(w, lo, h_sh, asrc_sh, adst_loc, idx_rows,
                              apair_chunk, ew_chunk, rows, acc, zacc,
                              sem, sem2)

      def do_block(src_v, dst_v, cursor):
        def sel_body(i, cur):
          sv = src_v[pl.ds(i * 16, 16)]
          dv = dst_v[pl.ds(i * 16, 16)]
          m = (dv >= lo) & (dv < lo + NLOC)
          c = jnp.cumsum(m.astype(jnp.int32))
          pos = jnp.where(m, cur + c - 1, B_E + 8)
          plsc.store_scatter(sel_pk, [pos], ((dv - lo) << 15) | sv)
          return cur + c[15]
        cnt = lax.fori_loop(0, B_E // 16, sel_body, jnp.int32(0))
        sel_pk[pl.ds(cnt, 16)] = sent16
        nch = (cnt + 15) // 16

        def wr(ci, carry):
          pltpu.async_copy(
              sel_pk.at[pl.ds(ci * 16, 16)],
              elist_hbm.at[pl.ds(pl.multiple_of(w * CAP + cursor + ci * 16, 8), 16)], sem_w)
          return carry
        lax.fori_loop(0, nch, wr, 0)

        def g_body(gi, carry):
          agg_chunk(sel_pk, gi * ROWC, cnt - gi * ROWC)
          return carry
        lax.fori_loop(0, (cnt + ROWC - 1) // ROWC, g_body, 0)

        def wrw(ci, carry):
          pltpu.make_async_copy(
              sel_pk.at[pl.ds(0, 16)],
              elist_hbm.at[pl.ds(w * CAP, 16)], sem_w).wait()
          return carry
        lax.fori_loop(0, nch, wrw, 0)
        return cursor + nch * 16

      pltpu.async_copy(src_hbm.at[pl.ds(0, B_E)], src_a, sem_s)
      pltpu.async_copy(dst_hbm.at[pl.ds(0, B_E)], dst_a, sem_d)

      def blk2_body(bb, cursor):
        base0 = 2 * bb * B_E
        base1 = base0 + B_E
        pltpu.make_async_copy(src_hbm.at[pl.ds(base0, B_E)], src_a,
                              sem_s).wait()
        pltpu.make_async_copy(dst_hbm.at[pl.ds(base0, B_E)], dst_a,
                              sem_d).wait()
        pltpu.async_copy(src_hbm.at[pl.ds(base1, B_E)], src_b, sem_s)
        pltpu.async_copy(dst_hbm.at[pl.ds(base1, B_E)], dst_b, sem_d)
        cursor = do_block(src_a, dst_a, cursor)
        pltpu.make_async_copy(src_hbm.at[pl.ds(base1, B_E)], src_b,
                              sem_s).wait()
        pltpu.make_async_copy(dst_hbm.at[pl.ds(base1, B_E)], dst_b,
                              sem_d).wait()

        @pl.when(bb < NBLK // 2 - 1)
        def _():
          base2 = base1 + B_E
          pltpu.async_copy(src_hbm.at[pl.ds(base2, B_E)], src_a, sem_s)
          pltpu.async_copy(dst_hbm.at[pl.ds(base2, B_E)], dst_a, sem_d)
        cursor = do_block(src_b, dst_b, cursor)
        return cursor
      total = lax.fori_loop(0, NBLK // 2, blk2_body, jnp.int32(0))

      cntb[...] = jnp.full((16,), total)
      pltpu.sync_copy(cntb, counts_hbm.at[pl.ds(w * 16, 16)])

      def fin(j, carry):
        invv = 1.0 / (zacc[j, :] + 1e-16)
        for f in range(d // 16):
          acc[j, pl.ds(f * 16, 16)] = acc[j, pl.ds(f * 16, 16)] * invv
        return carry
      lax.fori_loop(0, NLOC, fin, 0)
      pltpu.sync_copy(acc.at[pl.ds(0, NLOC)], out_hbm.at[pl.ds(lo, NLOC)])

    return k(src, dst, asrc, adst, h_pk)

  else:
    @functools.partial(
        pl.kernel, out_type=out_type, mesh=mesh,
        compiler_params=pltpu.CompilerParams(needs_layout_passes=False),
        scratch_types=scratch)
    def k(elist_hbm, counts_hbm, asrc_hbm, adst_hbm, hpk_hbm, out_hbm,
          h_sh, asrc_sh, adst_loc, idx_rows, apair_chunk, ew_chunk,
          rows, acc, zacc, cntb, sem, sem2, sem_s, sem_d, sem_w,
          sel_a, sel_b, idx_b, apair_b, rows_b, sem3, sem4):
      s_id = lax.axis_index("s")
      w = s_id * NC + lax.axis_index("c")
      lo = w * NLOC

      @pl.when(s_id < 5)
      def _():
        pltpu.sync_copy(hpk_hbm.at[pl.ds(s_id * 1000, 1000)],
                        h_sh.at[pl.ds(s_id * 1000, 1000)])

      @pl.when(s_id == 5)
      def _():
        pltpu.sync_copy(asrc_hbm, asrc_sh)

      pltpu.sync_copy(adst_hbm.at[pl.ds(lo, NLOC + 16)], adst_loc)
      pltpu.sync_copy(counts_hbm.at[pl.ds(w * 16, 16)], cntb)
      plsc.subcore_barrier()

      total = cntb[pl.ds(0, 16)][0]

      zeros16f = jnp.zeros((16,), jnp.float32)

      def zrow(j, carry):
        for f in range(d // 16):
          acc[j, pl.ds(f * 16, 16)] = zeros16f
        zacc[j, :] = zeros16f
        return carry
      lax.fori_loop(0, NLOC, zrow, 0)

      himask2 = jnp.full((16,), jnp.int32(-65536))
      bufs = [(idx_rows, rows, apair_chunk, sem, sem2),
              (idx_b, rows_b, apair_b, sem3, sem4)]

      def fill_issue(sel_ref, off, bi):
        ib, rb, ab, s1, s2 = bufs[bi]
        for q in range(1):
          sv = sel_ref[pl.ds(off + q * 16, 16)] & 32767
          ib[pl.ds(q * 16, 16)] = jnp.minimum(sv >> 1, N // 2 - 1)
        pltpu.async_copy(h_sh.at[ib], rb, s1)
        pltpu.async_copy(asrc_sh.at[ib], ab, s2)

      def compute_sub(sel_ref, off, navail, bi):
        ib, rb, ab, s1, s2 = bufs[bi]
        pltpu.make_async_copy(h_sh.at[ib], rb, s1).wait()
        pltpu.make_async_copy(asrc_sh.at[ib], ab, s2).wait()
        for q in range(1):
          pk = sel_ref[pl.ds(off + q * 16, 16)]
          raw = ab[pl.ds(q * 16, 16)]
          asr = jnp.where((pk & 1) == 1,
                          plsc.bitcast(raw & himask2, jnp.float32),
                          plsc.bitcast(raw << 16, jnp.float32))
          a = asr + plsc.load_gather(adst_loc,
                                     [jnp.minimum(pk >> 15, NLOC)])
          e = jnp.maximum(a, NEG_SLOPE * a)
          live = (pk & SENT) == 0
          ew_chunk[pl.ds(q * 16, 16)] = jnp.where(live, jnp.exp(e), 0.0)
        kmax = jnp.minimum(16, navail)

        def k_body(ki, c2):
          s = sel_ref[pl.ds(off + ki, 16)][0]
          j = s >> 15
          colbase = (s & 1) * 64
          sw = ew_chunk[pl.ds(ki, 16)][0]
          for f in range(nfb):
            raw = rb[ki, pl.ds(colbase + f * 16, 16)]
            evn = plsc.bitcast(raw << 16, jnp.float32)
            odd = plsc.bitcast(raw & himask2, jnp.float32)
            acc[j, pl.ds(f * 32, 16)] = (
                acc[j, pl.ds(f * 32, 16)] + evn * sw)
            acc[j, pl.ds(f * 32 + 16, 16)] = (
                acc[j, pl.ds(f * 32 + 16, 16)] + odd * sw)
          zacc[j, :] = zacc[j, :] + sw
          return c2
        lax.fori_loop(0, kmax, k_body, 0)

      NSUB = CH // 16

      def do_chunk(sel_ref, base):
        fill_issue(sel_ref, 0, 0)
        for gi in range(NSUB):
          if gi < NSUB - 1:
            fill_issue(sel_ref, (gi + 1) * 16, (gi + 1) % 2)
          compute_sub(sel_ref, gi * 16, total - base - gi * 16, gi % 2)

      pltpu.async_copy(elist_hbm.at[pl.ds(w * CAP, CH)], sel_a.at[pl.ds(0, CH)], sem_s)
      npair = (total + 2 * CH - 1) // (2 * CH)

      def ch2_body(cc, carry):
        base0 = 2 * cc * CH
        base1 = base0 + CH
        pltpu.make_async_copy(elist_hbm.at[pl.ds(w * CAP, CH)], sel_a.at[pl.ds(0, CH)],
                              sem_s).wait()
        b1c = jnp.minimum(base1, CAP - CH)
        pltpu.async_copy(elist_hbm.at[pl.ds(pl.multiple_of(w * CAP + b1c, 8), CH)], sel_b.at[pl.ds(0, CH)], sem_d)
        do_chunk(sel_a, base0)
        pltpu.make_async_copy(elist_hbm.at[pl.ds(w * CAP, CH)], sel_b.at[pl.ds(0, CH)],
                              sem_d).wait()

        @pl.when(cc < npair - 1)
        def _():
          base2 = jnp.minimum(base1 + CH, CAP - CH)
          pltpu.async_copy(elist_hbm.at[pl.ds(pl.multiple_of(w * CAP + base2, 8), CH)],
                           sel_a.at[pl.ds(0, CH)], sem_s)
        do_chunk(sel_b, base1)
        return carry
      lax.fori_loop(0, npair, ch2_body, 0)

      def fin(j, carry):
        invv = 1.0 / (zacc[j, :] + 1e-16)
        for f in range(d // 16):
          acc[j, pl.ds(f * 16, 16)] = acc[j, pl.ds(f * 16, 16)] * invv
        return carry
      lax.fori_loop(0, NLOC, fin, 0)
      pltpu.sync_copy(acc.at[pl.ds(0, NLOC)], out_hbm.at[pl.ds(lo, NLOC)])

    return k(elist, counts, asrc, adst, h_pk)


def _pack_rows(h_perm):
  hb = h_perm.astype(jnp.bfloat16).reshape(N // 2, 128, 2)
  return jax.lax.bitcast_convert_type(hb, jnp.int32)


def _pack_alpha(asrc):
  ab = asrc.astype(jnp.bfloat16).reshape(N // 2, 2)
  return jax.lax.bitcast_convert_type(ab, jnp.int32)


def kernel(in_feat, g, W1, a_src1, a_dst1, W2, a_src2, a_dst2):
  src = g[0]
  dst = g[1]
  perm = jnp.asarray(PERM)

  W1p = W1[:, perm]
  a21 = jnp.stack([a_src1, a_dst1], axis=1)[perm]
  h1, al1 = _tc_proj(in_feat, W1p, a21, apply_relu=False)
  adst1_pad = jnp.pad(al1[:, 1], (0, N_PAD + 16 - N))
  out1p, elist, counts = _sc_layer(_pack_alpha(al1[:, 0]), adst1_pad,
                                   _pack_rows(h1), 128, emit=True,
                                   src=src, dst=dst)
  out1 = out1p[:N]

  d2 = W2.shape[1]
  W2p = jnp.pad(W2, ((0, 0), (0, 128 - d2)))[:, perm]
  a22 = jnp.pad(jnp.stack([a_src2, a_dst2], axis=1),
                ((0, 128 - d2), (0, 0)))[perm]
  h2, al2 = _tc_proj(out1, W2p, a22, apply_relu=True)
  adst2_pad = jnp.pad(al2[:, 1], (0, N_PAD + 16 - N))
  out2 = _sc_layer(_pack_alpha(al2[:, 0]), adst2_pad, _pack_rows(h2),
                   d2, emit=False, elist=elist, counts=counts)[:N]
  return out2
